# baseline (speedup 1.0000x reference)
import jax
import jax.numpy as jnp
from jax import lax
from jax.experimental import pallas as pl
from jax.experimental.pallas import tpu as pltpu
from jax.experimental.pallas import tpu_sc as plsc

_NC = 2
_NS = 16
_L = 16
_NW = _NC * _NS

_BATCH = 16384
_DIM = 32
_ROWS = 1000000
_GRP = 4096
_NSTEP = 62
_B = 488 * 512
_KSTRIDE = _B // _GRP
_LROWS = _NSTEP * _GRP
_BPW = _BATCH // _NW
_CH = 128
_NCHUNK = _BPW // _CH


def _pack_body(u0, u1, u2, u3, out_u):
    rows = lax.broadcasted_iota(jnp.int32, (128, 128), 0)
    cols = lax.broadcasted_iota(jnp.int32, (128, 128), 1)
    eye = jnp.where(rows == cols, jnp.float32(1.0), jnp.float32(0.0))

    def tr4(r0, r1, r2, r3):
        x = jnp.concatenate([r0[...], r1[...], r2[...], r3[...]], axis=0)
        return lax.dot_general(x, eye, (((0,), (0,)), ((), ())),
                               preferred_element_type=jnp.float32)

    out_u[...] = tr4(u0, u1, u2, u3)


def _pack_table_tc(ut):
    in_specs = [
        pl.BlockSpec((_DIM, _GRP), lambda i, k=k: (0, i + k * _KSTRIDE))
        for k in range(4)
    ]
    return pl.pallas_call(
        _pack_body,
        grid=(_NSTEP,),
        in_specs=in_specs,
        out_specs=pl.BlockSpec((_GRP, 128), lambda i: (i, 0)),
        out_shape=jax.ShapeDtypeStruct((_LROWS, 128), jnp.float32),
    )(ut, ut, ut, ut)


_PKW = 256
_NSLAB = 253952 // _PKW
_TAILSID = (_ROWS - 3 * _B - 64) // _PKW


def _pack_m_body(mt_hbm, tail_hbm, pm_hbm, slabs, buf, tailb, sem, osem):
    wid = lax.axis_index("s") * _NC + lax.axis_index("c")
    pltpu.sync_copy(tail_hbm, tailb)
    lane = lax.iota(jnp.int32, _L)

    def xpose(src_ref, k, ncols):
        def g_body(g, _):
            cvec = g * _L + lane
            for d in range(_DIM):
                v = plsc.load_gather(src_ref, [jnp.full((_L,), d, jnp.int32),
                                               cvec])
                plsc.store_scatter(buf, [cvec,
                                         jnp.full((_L,), 32 * k + d,
                                                  jnp.int32)], v)
            return _
        lax.fori_loop(0, ncols // _L, g_body, 0, unroll=False)

    def fetch(k, sid, slot):
        col = pl.multiple_of((sid * _PKW + k * _B), _PKW)
        return pltpu.async_copy(mt_hbm.at[:, pl.ds(col, _PKW)],
                                slabs.at[slot], sem.at[slot])

    def slab_body(s, carry):
        sid = s * _NW + wid
        cp = fetch(0, sid, 0)
        for k in range(3):
            nxt = fetch(k + 1, sid, (k + 1) % 2) if k < 2 else None
            cp.wait()
            xpose(slabs.at[k % 2], k, _PKW)
            cp = nxt
        @pl.when(sid <= _TAILSID - 1)
        def _full3():
            cp3 = fetch(3, sid, 1)
            cp3.wait()
            xpose(slabs.at[1], 3, _PKW)

        @pl.when(sid == _TAILSID)
        def _tail3():
            xpose(tailb, 3, 64)

        pltpu.sync_copy(buf, pm_hbm.at[pl.ds(sid * _PKW, _PKW)])
        return carry

    lax.fori_loop(0, _NSLAB // _NW, slab_body, 0, unroll=False)


def _pack_table_sc(mt, tail_m):
    mesh = plsc.VectorSubcoreMesh(core_axis_name="c", subcore_axis_name="s",
                                  num_cores=_NC, num_subcores=_NS)
    return pl.kernel(
        _pack_m_body,
        out_type=jax.ShapeDtypeStruct((_LROWS, 128), jnp.float32),
        mesh=mesh,
        compiler_params=pltpu.CompilerParams(needs_layout_passes=False),
        scratch_types=[
            pltpu.VMEM((2, _DIM, _PKW), jnp.float32),
            pltpu.VMEM((_PKW, 128), jnp.float32),
            pltpu.VMEM((_DIM, 64), jnp.float32),
            pltpu.SemaphoreType.DMA((2,)),
            pltpu.SemaphoreType.DMA,
        ],
    )(mt, tail_m)


def _rsqrt_newton(s):
    i = plsc.bitcast(s, jnp.int32)
    y = plsc.bitcast(jnp.int32(0x5F3759DF) - lax.shift_right_logical(i, 1),
                     jnp.float32)
    half_s = 0.5 * s
    for _ in range(3):
        y = y * (1.5 - half_s * y * y)
    return y


def _scale(sq):
    return jnp.minimum(jnp.float32(1.0),
                       _rsqrt_newton(jnp.maximum(sq, jnp.float32(1e-14))))


def _als_body(pu_hbm, pm_hbm, usr_hbm, movie_hbm, out_hbm,
              iu_raw, im_raw, iu4, im4, off_u, off_m,
              ru, rm, out_v, sem):
    wid = lax.axis_index("s") * _NC + lax.axis_index("c")
    base = wid * _BPW

    pltpu.sync_copy(usr_hbm.at[pl.ds(base, _BPW)], iu_raw)
    pltpu.sync_copy(movie_hbm.at[pl.ds(base, _BPW)], im_raw)

    blocks_per_chunk = _CH // _L
    b1, b2, b3 = jnp.int32(_B), jnp.int32(2 * _B), jnp.int32(3 * _B)
    one, zero = jnp.int32(1), jnp.int32(0)

    def group_of(v):
        k = jnp.where(v >= b1, one, zero)
        k = k + jnp.where(v >= b2, one, zero)
        return k + jnp.where(v >= b3, one, zero)

    def prep(c, _):
        sl = pl.ds(c * _L, _L)
        j = c // blocks_per_chunk
        r = (c % blocks_per_chunk) * _L
        vu = iu_raw[sl]
        ku = group_of(vu)
        iu4[j, pl.ds(r, _L)] = vu - ku * b1
        off_u[sl] = ku * _DIM
        vm = im_raw[sl]
        km = group_of(vm)
        im4[j, pl.ds(r, _L)] = vm - km * b1
        off_m[sl] = km * _DIM
        return _

    lax.fori_loop(0, _BPW // _L, prep, 0, unroll=False)

    lane = lax.iota(jnp.int32, _L)

    def gather(j, buf_slot):
        cu = pltpu.async_copy(pu_hbm.at[iu4.at[j]], ru.at[buf_slot],
                              sem.at[buf_slot])
        cm = pltpu.async_copy(pm_hbm.at[im4.at[j]], rm.at[buf_slot],
                              sem.at[buf_slot])
        return cu, cm

    cps = {0: gather(0, 0)}
    for j in range(_NCHUNK):
        if j + 1 < _NCHUNK:
            cps[j + 1] = gather(j + 1, (j + 1) % 2)
        cu, cm = cps.pop(j)
        cu.wait()
        cm.wait()
        slot = j % 2

        def block(blk, _, j=j, slot=slot):
            row = blk * _L + lane
            bsl = pl.ds(j * _CH + blk * _L, _L)
            du = off_u[bsl]
            dm = off_m[bsl]
            su = jnp.zeros((_L,), jnp.float32)
            sm = jnp.zeros((_L,), jnp.float32)
            dp = jnp.zeros((_L,), jnp.float32)
            for d in range(_DIM):
                uv = plsc.load_gather(ru.at[slot], [row, du + d])
                mv = plsc.load_gather(rm.at[slot], [row, dm + d])
                su = su + uv * uv
                sm = sm + mv * mv
                dp = dp + uv * mv
            x = dp * _scale(su) * _scale(sm)
            out_v[bsl] = 1.0 / (1.0 + jnp.exp(-x))
            return _

        lax.fori_loop(0, _CH // _L, block, 0, unroll=False)

    pltpu.sync_copy(out_v, out_hbm.at[pl.ds(base, _BPW)])


def _als_sc(pu, pm, usr, movie):
    mesh = plsc.VectorSubcoreMesh(core_axis_name="c", subcore_axis_name="s",
                                  num_cores=_NC, num_subcores=_NS)
    return pl.kernel(
        _als_body,
        out_type=jax.ShapeDtypeStruct((_BATCH,), jnp.float32),
        mesh=mesh,
        compiler_params=pltpu.CompilerParams(needs_layout_passes=False),
        scratch_types=[
            pltpu.VMEM((_BPW,), jnp.int32),
            pltpu.VMEM((_BPW,), jnp.int32),
            pltpu.VMEM((_NCHUNK, _CH), jnp.int32),
            pltpu.VMEM((_NCHUNK, _CH), jnp.int32),
            pltpu.VMEM((_BPW,), jnp.int32),
            pltpu.VMEM((_BPW,), jnp.int32),
            pltpu.VMEM((2, _CH, 128), jnp.float32),
            pltpu.VMEM((2, _CH, 128), jnp.float32),
            pltpu.VMEM((_BPW,), jnp.float32),
            pltpu.SemaphoreType.DMA((2,)),
        ],
    )(pu, pm, usr, movie)


@jax.jit
def _als(usr, movie, usr_emd, movie_emd):
    mt = movie_emd.T
    tail_m = lax.slice(mt, (0, _ROWS - 64), (_DIM, _ROWS))
    pm = _pack_table_sc(mt, tail_m)
    pu = _pack_table_tc(usr_emd.T)
    return _als_sc(pu, pm, usr, movie)


def kernel(usr, movie, usr_emd, movie_emd):
    return _als(usr, movie, usr_emd, movie_emd)


# speedup vs baseline: 3.8004x; 3.8004x over previous
import jax
import jax.numpy as jnp
from jax import lax
from jax.experimental import pallas as pl
from jax.experimental.pallas import tpu as pltpu
from jax.experimental.pallas import tpu_sc as plsc

_NC = 2
_NS = 16
_L = 16
_NW = _NC * _NS

_BATCH = 16384
_DIM = 32
_ROWS = 1000000
_GRP = 4096
_NSTEP = 62
_B = 488 * 512
_KSTRIDE = _B // _GRP
_LROWS = _NSTEP * _GRP
_BPW = _BATCH // _NW
_CH = 128
_NCHUNK = _BPW // _CH


def _pack_body(u0, u1, u2, u3, m0, m1, m2, m3, out_u, out_m):
    rows = lax.broadcasted_iota(jnp.int32, (128, 128), 0)
    cols = lax.broadcasted_iota(jnp.int32, (128, 128), 1)
    eye = jnp.where(rows == cols, jnp.float32(1.0), jnp.float32(0.0))

    def tr4(r0, r1, r2, r3):
        x = jnp.concatenate([r0[...], r1[...], r2[...], r3[...]], axis=0)
        return lax.dot_general(x, eye, (((0,), (0,)), ((), ())),
                               preferred_element_type=jnp.float32)

    out_u[...] = tr4(u0, u1, u2, u3)
    out_m[...] = tr4(m0, m1, m2, m3)


def _pack_tables(ut, mt):
    in_specs = [
        pl.BlockSpec((_DIM, _GRP), lambda i, k=k: (0, i + k * _KSTRIDE))
        for k in range(4)
    ] * 2
    return pl.pallas_call(
        _pack_body,
        grid=(_NSTEP,),
        in_specs=in_specs,
        out_specs=[
            pl.BlockSpec((_GRP, 128), lambda i: (i, 0)),
            pl.BlockSpec((_GRP, 128), lambda i: (i, 0)),
        ],
        out_shape=[
            jax.ShapeDtypeStruct((_LROWS, 128), jnp.float32),
            jax.ShapeDtypeStruct((_LROWS, 128), jnp.float32),
        ],
    )(ut, ut, ut, ut, mt, mt, mt, mt)


def _rsqrt_newton(s):
    i = plsc.bitcast(s, jnp.int32)
    y = plsc.bitcast(jnp.int32(0x5F3759DF) - lax.shift_right_logical(i, 1),
                     jnp.float32)
    half_s = 0.5 * s
    for _ in range(3):
        y = y * (1.5 - half_s * y * y)
    return y


def _scale(sq):
    return jnp.minimum(jnp.float32(1.0),
                       _rsqrt_newton(jnp.maximum(sq, jnp.float32(1e-14))))


def _als_body(pu_hbm, pm_hbm, usr_hbm, movie_hbm, out_hbm,
              iu_raw, im_raw, iu4, im4, off_u, off_m,
              ru, rm, out_v, sem):
    wid = lax.axis_index("s") * _NC + lax.axis_index("c")
    base = wid * _BPW

    pltpu.sync_copy(usr_hbm.at[pl.ds(base, _BPW)], iu_raw)
    pltpu.sync_copy(movie_hbm.at[pl.ds(base, _BPW)], im_raw)

    blocks_per_chunk = _CH // _L
    b1, b2, b3 = jnp.int32(_B), jnp.int32(2 * _B), jnp.int32(3 * _B)
    one, zero = jnp.int32(1), jnp.int32(0)

    def group_of(v):
        k = jnp.where(v >= b1, one, zero)
        k = k + jnp.where(v >= b2, one, zero)
        return k + jnp.where(v >= b3, one, zero)

    def prep(c, _):
        sl = pl.ds(c * _L, _L)
        j = c // blocks_per_chunk
        r = (c % blocks_per_chunk) * _L
        vu = iu_raw[sl]
        ku = group_of(vu)
        iu4[j, pl.ds(r, _L)] = vu - ku * b1
        off_u[sl] = ku * _DIM
        vm = im_raw[sl]
        km = group_of(vm)
        im4[j, pl.ds(r, _L)] = vm - km * b1
        off_m[sl] = km * _DIM
        return _

    lax.fori_loop(0, _BPW // _L, prep, 0, unroll=False)

    lane = lax.iota(jnp.int32, _L)

    def gather(j, buf_slot):
        cu = pltpu.async_copy(pu_hbm.at[iu4.at[j]], ru.at[buf_slot],
                              sem.at[buf_slot])
        cm = pltpu.async_copy(pm_hbm.at[im4.at[j]], rm.at[buf_slot],
                              sem.at[buf_slot])
        return cu, cm

    cps = {0: gather(0, 0)}
    for j in range(_NCHUNK):
        if j + 1 < _NCHUNK:
            cps[j + 1] = gather(j + 1, (j + 1) % 2)
        cu, cm = cps.pop(j)
        cu.wait()
        cm.wait()
        slot = j % 2

        def block(blk, _, j=j, slot=slot):
            row = blk * _L + lane
            bsl = pl.ds(j * _CH + blk * _L, _L)
            du = off_u[bsl]
            dm = off_m[bsl]
            su = jnp.zeros((_L,), jnp.float32)
            sm = jnp.zeros((_L,), jnp.float32)
            dp = jnp.zeros((_L,), jnp.float32)
            for d in range(_DIM):
                uv = plsc.load_gather(ru.at[slot], [row, du + d])
                mv = plsc.load_gather(rm.at[slot], [row, dm + d])
                su = su + uv * uv
                sm = sm + mv * mv
                dp = dp + uv * mv
            x = dp * _scale(su) * _scale(sm)
            out_v[bsl] = 1.0 / (1.0 + jnp.exp(-x))
            return _

        lax.fori_loop(0, _CH // _L, block, 0, unroll=False)

    pltpu.sync_copy(out_v, out_hbm.at[pl.ds(base, _BPW)])


def _als_sc(pu, pm, usr, movie):
    mesh = plsc.VectorSubcoreMesh(core_axis_name="c", subcore_axis_name="s",
                                  num_cores=_NC, num_subcores=_NS)
    return pl.kernel(
        _als_body,
        out_type=jax.ShapeDtypeStruct((_BATCH,), jnp.float32),
        mesh=mesh,
        compiler_params=pltpu.CompilerParams(needs_layout_passes=False),
        scratch_types=[
            pltpu.VMEM((_BPW,), jnp.int32),
            pltpu.VMEM((_BPW,), jnp.int32),
            pltpu.VMEM((_NCHUNK, _CH), jnp.int32),
            pltpu.VMEM((_NCHUNK, _CH), jnp.int32),
            pltpu.VMEM((_BPW,), jnp.int32),
            pltpu.VMEM((_BPW,), jnp.int32),
            pltpu.VMEM((2, _CH, 128), jnp.float32),
            pltpu.VMEM((2, _CH, 128), jnp.float32),
            pltpu.VMEM((_BPW,), jnp.float32),
            pltpu.SemaphoreType.DMA((2,)),
        ],
    )(pu, pm, usr, movie)


@jax.jit
def _als(usr, movie, usr_emd, movie_emd):
    pu, pm = _pack_tables(usr_emd.T, movie_emd.T)
    return _als_sc(pu, pm, usr, movie)


def kernel(usr, movie, usr_emd, movie_emd):
    return _als(usr, movie, usr_emd, movie_emd)
